# Initial kernel scaffold; baseline (speedup 1.0000x reference)
import jax
import jax.numpy as jnp
from jax.experimental import pallas as pl


def kernel(x, edge_index, edge_attr, W1, Wih1, Whh1, bih1, bhh1, W2, Wih2, Whh2, bih2, bhh2, lin1_W, lin1_b, lin2_W, lin2_b, bn1_g, bn1_b, bn2_g, bn2_b, bn3_g, bn3_b):
    raise NotImplementedError("write your pallas kernel here")


# speedup vs baseline: 5.2149x; 5.2149x over previous
import functools

import jax
import jax.numpy as jnp
from jax import lax
from jax.experimental import pallas as pl
from jax.experimental.pallas import tpu as pltpu
from jax.experimental.pallas import tpu_sc as plsc

_N = 10000
_E = 320000
_H = 128

_BLK = 128
_WORKERS = 32
_NBLK = 79
_EPW = _NBLK * _BLK
_EPAD = _WORKERS * _EPW
_RFULL = 632
_RLAST = _N - 15 * _RFULL

_F32 = jnp.float32
_HI = lax.Precision.HIGHEST


def _seg_body(m_hbm, src_hbm, dst_hbm, ew_hbm, out_hbm,
              srcs_v, dst0, dst1, ew0, ew1, rows0, rows1, agg_sh,
              semg0, semg1, semi0, semi1):
    cid = lax.axis_index("c")
    sid = lax.axis_index("s")
    wid = cid * 16 + sid

    pltpu.async_copy(src_hbm.at[wid], srcs_v, semi0)

    rows_v = rows0

    def zrow(i, carry):
        for j in range(8):
            rows_v[i, pl.ds(16 * j, 16)] = jnp.zeros((16,), _F32)
        return carry
    lax.fori_loop(0, _BLK, zrow, None)
    row0 = sid * _RFULL

    @pl.when(sid < 15)
    def _():
        for k in range(_RFULL // _BLK):
            pltpu.sync_copy(rows_v, agg_sh.at[pl.ds(row0 + k * _BLK, _BLK)])
        rem = _RFULL % _BLK
        pltpu.sync_copy(rows_v.at[pl.ds(0, rem)],
                        agg_sh.at[pl.ds(row0 + _RFULL - rem, rem)])

    @pl.when(sid == 15)
    def _():
        for k in range(_RLAST // _BLK):
            pltpu.sync_copy(rows_v, agg_sh.at[pl.ds(row0 + k * _BLK, _BLK)])
        rem = _RLAST % _BLK
        pltpu.sync_copy(rows_v.at[pl.ds(0, rem)],
                        agg_sh.at[pl.ds(row0 + _RLAST - rem, rem)])

    pltpu.make_async_copy(src_hbm.at[wid], srcs_v, semi0).wait()

    bufs = ((rows0, dst0, ew0, semg0, semi0), (rows1, dst1, ew1, semg1, semi1))

    def _fetch(b, k):
        rows, dstb, ewb, semg, semi = bufs[k]
        pltpu.async_copy(m_hbm.at[srcs_v.at[b]], rows, semg)
        pltpu.async_copy(dst_hbm.at[wid, pl.ds(b, 1)], dstb, semi)
        pltpu.async_copy(ew_hbm.at[wid, pl.ds(b, 1)], ewb, semi)

    def _process(b, k):
        rows, dstb, ewb, semg, semi = bufs[k]
        pltpu.make_async_copy(m_hbm.at[srcs_v.at[b]], rows, semg).wait()
        pltpu.make_async_copy(dst_hbm.at[wid, pl.ds(b, 1)], dstb, semi).wait()
        pltpu.make_async_copy(ew_hbm.at[wid, pl.ds(b, 1)], ewb, semi).wait()

        def scale(g, c2):
            wv = ewb[0, pl.ds(g * 16, 16)]
            for i in range(16):
                w = jnp.broadcast_to(wv[i], (16,))
                r = g * 16 + i
                for j in range(8):
                    sl = pl.ds(16 * j, 16)
                    rows[r, sl] = rows[r, sl] * w
            return c2
        lax.fori_loop(0, _BLK // 16, scale, None)

        pltpu.sync_copy(rows, agg_sh.at[dstb.at[0]], add=True)

    _fetch(0, 0)
    _fetch(1, 1)
    plsc.subcore_barrier()

    def pair(t, carry):
        b = 2 * t
        _process(b, 0)

        @pl.when(b + 2 < _NBLK)
        def _():
            _fetch(b + 2, 0)

        _process(b + 1, 1)

        @pl.when(b + 3 < _NBLK)
        def _():
            _fetch(b + 3, 1)

        return carry

    lax.fori_loop(0, (_NBLK - 1) // 2, pair, None)
    _process(_NBLK - 1, 0)
    plsc.subcore_barrier()

    @pl.when(sid < 15)
    def _():
        pltpu.sync_copy(agg_sh.at[pl.ds(row0, _RFULL)],
                        out_hbm.at[cid, pl.ds(row0, _RFULL)])

    @pl.when(sid == 15)
    def _():
        pltpu.sync_copy(agg_sh.at[pl.ds(row0, _RLAST)],
                        out_hbm.at[cid, pl.ds(row0, _RLAST)])


_seg = functools.partial(
    pl.kernel,
    out_type=jax.ShapeDtypeStruct((2, _N, _H), _F32),
    scratch_types=[
        pltpu.VMEM((_NBLK, _BLK), jnp.int32),
        pltpu.VMEM((1, _BLK), jnp.int32),
        pltpu.VMEM((1, _BLK), jnp.int32),
        pltpu.VMEM((1, _BLK), _F32),
        pltpu.VMEM((1, _BLK), _F32),
        pltpu.VMEM((_BLK, _H), _F32),
        pltpu.VMEM((_BLK, _H), _F32),
        pltpu.VMEM_SHARED((_N, _H), _F32),
        pltpu.SemaphoreType.DMA,
        pltpu.SemaphoreType.DMA,
        pltpu.SemaphoreType.DMA,
        pltpu.SemaphoreType.DMA,
    ],
    mesh=plsc.VectorSubcoreMesh(core_axis_name="c", subcore_axis_name="s"),
)(_seg_body)


_RB = 2000
_G = _N // _RB


def _dotT(a, w):
    return lax.dot_general(a, w, (((1,), (1,)), ((), ())),
                           precision=_HI, preferred_element_type=_F32)


def _blk(shape, idx):
    return pl.BlockSpec(shape, idx)


def _pre_body(x_ref, w_ref, whh_ref, bhh_ref, m_ref, gh_ref):
    xv = x_ref[...]
    m_ref[...] = lax.dot_general(xv, w_ref[...], (((1,), (0,)), ((), ())),
                                 precision=_HI, preferred_element_type=_F32)
    gh_ref[...] = _dotT(xv, whh_ref[...]) + bhh_ref[...]


_pre = pl.pallas_call(
    _pre_body,
    grid=(_G,),
    in_specs=[_blk((_RB, _H), lambda i: (i, 0)),
              _blk((_H, _H), lambda i: (0, 0)),
              _blk((3 * _H, _H), lambda i: (0, 0)),
              _blk((1, 3 * _H), lambda i: (0, 0))],
    out_specs=[_blk((_RB, _H), lambda i: (i, 0)),
               _blk((_RB, 3 * _H), lambda i: (i, 0))],
    out_shape=[jax.ShapeDtypeStruct((_N, _H), _F32),
               jax.ShapeDtypeStruct((_N, 3 * _H), _F32)],
)


def _gru_body(parts_ref, gh_ref, xin_ref, wih_ref, bih_ref, hr_ref,
              stats_ref):
    agg = parts_ref[0] + parts_ref[1]
    gi = _dotT(agg, wih_ref[...]) + bih_ref[...]
    gh = gh_ref[...]
    r = jax.nn.sigmoid(gi[:, :_H] + gh[:, :_H])
    z = jax.nn.sigmoid(gi[:, _H:2 * _H] + gh[:, _H:2 * _H])
    n = jnp.tanh(gi[:, 2 * _H:] + r * gh[:, 2 * _H:])
    h = (1.0 - z) * n + z * xin_ref[...]
    h = jnp.maximum(h, 0.0)
    hr_ref[...] = h

    @pl.when(pl.program_id(0) == 0)
    def _():
        stats_ref[...] = jnp.zeros((2, _H), _F32)

    s1 = jnp.sum(h, axis=0, keepdims=True)
    s2 = jnp.sum(h * h, axis=0, keepdims=True)
    stats_ref[...] += jnp.concatenate([s1, s2], axis=0)


_gru = pl.pallas_call(
    _gru_body,
    grid=(_G,),
    in_specs=[_blk((2, _RB, _H), lambda i: (0, i, 0)),
              _blk((_RB, 3 * _H), lambda i: (i, 0)),
              _blk((_RB, _H), lambda i: (i, 0)),
              _blk((3 * _H, _H), lambda i: (0, 0)),
              _blk((1, 3 * _H), lambda i: (0, 0))],
    out_specs=[_blk((_RB, _H), lambda i: (i, 0)),
               _blk((2, _H), lambda i: (0, 0))],
    out_shape=[jax.ShapeDtypeStruct((_N, _H), _F32),
               jax.ShapeDtypeStruct((2, _H), _F32)],
)


def _bn_from_stats(h, stats_ref, g_ref, b_ref):
    mu = stats_ref[0:1, :] * (1.0 / _N)
    var = stats_ref[1:2, :] * (1.0 / _N) - mu * mu
    return (h - mu) / jnp.sqrt(var + 1e-5) * g_ref[...] + b_ref[...]


def _bnpre_body(hr_ref, stats_ref, g_ref, b_ref, w_ref, whh_ref, bhh_ref,
                h1_ref, m_ref, gh_ref):
    h1 = _bn_from_stats(hr_ref[...], stats_ref, g_ref, b_ref)
    h1_ref[...] = h1
    m_ref[...] = lax.dot_general(h1, w_ref[...], (((1,), (0,)), ((), ())),
                                 precision=_HI, preferred_element_type=_F32)
    gh_ref[...] = _dotT(h1, whh_ref[...]) + bhh_ref[...]


_bnpre = pl.pallas_call(
    _bnpre_body,
    grid=(_G,),
    in_specs=[_blk((_RB, _H), lambda i: (i, 0)),
              _blk((2, _H), lambda i: (0, 0)),
              _blk((1, _H), lambda i: (0, 0)),
              _blk((1, _H), lambda i: (0, 0)),
              _blk((_H, _H), lambda i: (0, 0)),
              _blk((3 * _H, _H), lambda i: (0, 0)),
              _blk((1, 3 * _H), lambda i: (0, 0))],
    out_specs=[_blk((_RB, _H), lambda i: (i, 0)),
               _blk((_RB, _H), lambda i: (i, 0)),
               _blk((_RB, 3 * _H), lambda i: (i, 0))],
    out_shape=[jax.ShapeDtypeStruct((_N, _H), _F32),
               jax.ShapeDtypeStruct((_N, _H), _F32),
               jax.ShapeDtypeStruct((_N, 3 * _H), _F32)],
)


def _bnlin_body(hr_ref, stats_ref, g_ref, b_ref, l1w_ref, l1b_ref, t_ref,
                stats3_ref):
    h = _bn_from_stats(hr_ref[...], stats_ref, g_ref, b_ref)
    t = jnp.maximum(_dotT(h, l1w_ref[...]) + l1b_ref[...], 0.0)
    t_ref[...] = t

    @pl.when(pl.program_id(0) == 0)
    def _():
        stats3_ref[...] = jnp.zeros((2, _H), _F32)

    s1 = jnp.sum(t, axis=0, keepdims=True)
    s2 = jnp.sum(t * t, axis=0, keepdims=True)
    stats3_ref[...] += jnp.concatenate([s1, s2], axis=0)


_bnlin = pl.pallas_call(
    _bnlin_body,
    grid=(_G,),
    in_specs=[_blk((_RB, _H), lambda i: (i, 0)),
              _blk((2, _H), lambda i: (0, 0)),
              _blk((1, _H), lambda i: (0, 0)),
              _blk((1, _H), lambda i: (0, 0)),
              _blk((_H, _H), lambda i: (0, 0)),
              _blk((1, _H), lambda i: (0, 0))],
    out_specs=[_blk((_RB, _H), lambda i: (i, 0)),
               _blk((2, _H), lambda i: (0, 0))],
    out_shape=[jax.ShapeDtypeStruct((_N, _H), _F32),
               jax.ShapeDtypeStruct((2, _H), _F32)],
)


def _out_body(t_ref, stats_ref, g_ref, b_ref, l2w_ref, out_ref):
    h = _bn_from_stats(t_ref[...], stats_ref, g_ref, b_ref)
    out_ref[...] = _dotT(h, l2w_ref[...])


_outk = pl.pallas_call(
    _out_body,
    grid=(_G,),
    in_specs=[_blk((_RB, _H), lambda i: (i, 0)),
              _blk((2, _H), lambda i: (0, 0)),
              _blk((1, _H), lambda i: (0, 0)),
              _blk((1, _H), lambda i: (0, 0)),
              _blk((1, _H), lambda i: (0, 0))],
    out_specs=[_blk((_RB, 1), lambda i: (i, 0))],
    out_shape=[jax.ShapeDtypeStruct((_N, 1), _F32)],
)


def kernel(x, edge_index, edge_attr, W1, Wih1, Whh1, bih1, bhh1, W2, Wih2,
           Whh2, bih2, bhh2, lin1_W, lin1_b, lin2_W, lin2_b, bn1_g, bn1_b,
           bn2_g, bn2_b, bn3_g, bn3_b):
    pad = _EPAD - _E
    src = jnp.pad(edge_index[0], (0, pad)).reshape(_WORKERS, _NBLK, _BLK)
    dst = jnp.pad(edge_index[1], (0, pad)).reshape(_WORKERS, _NBLK, _BLK)
    ew = jnp.pad(edge_attr, (0, pad)).reshape(_WORKERS, _NBLK, _BLK)

    m1, gh1 = _pre(x, W1, Whh1, bhh1.reshape(1, -1))
    parts1 = _seg(m1, src, dst, ew)
    hr1, stats1 = _gru(parts1, gh1, x, Wih1, bih1.reshape(1, -1))

    h1, m2, gh2 = _bnpre(hr1, stats1, bn1_g.reshape(1, -1),
                         bn1_b.reshape(1, -1), W2, Whh2,
                         bhh2.reshape(1, -1))
    parts2 = _seg(m2, src, dst, ew)
    hr2, stats2 = _gru(parts2, gh2, h1, Wih2, bih2.reshape(1, -1))

    t, stats3 = _bnlin(hr2, stats2, bn2_g.reshape(1, -1),
                       bn2_b.reshape(1, -1), lin1_W, lin1_b.reshape(1, -1))
    (out,) = _outk(t, stats3, bn3_g.reshape(1, -1), bn3_b.reshape(1, -1),
                   lin2_W)
    return out.reshape(-1) + lin2_b[0]


# speedup vs baseline: 5.5610x; 1.0664x over previous
import functools

import jax
import jax.numpy as jnp
from jax import lax
from jax.experimental import pallas as pl
from jax.experimental.pallas import tpu as pltpu
from jax.experimental.pallas import tpu_sc as plsc

_N = 10000
_E = 320000
_H = 128

_BLK = 128
_WORKERS = 32
_NB0 = 107
_NB1 = 51
_NBMAX = _NB0
_RFULL = 632
_RLAST = _N - 15 * _RFULL

_F32 = jnp.float32
_HI = lax.Precision.HIGHEST


def _seg_body(m_hbm, src_hbm, dst_hbm, ew_hbm, out_hbm,
              srcs_v, dst0, dst1, ew0, ew1, rows0, rows1, agg_sh,
              semg0, semg1, semi0, semi1):
    cid = lax.axis_index("c")
    sid = lax.axis_index("s")
    wid = cid * 16 + sid
    nblk = jnp.where(cid == 0, _NB0, _NB1)

    with jax.named_scope("seg_slab"):
        pltpu.async_copy(src_hbm.at[wid], srcs_v, semi0)

    rows_v = rows0

    def zrow(i, carry):
        for j in range(8):
            rows_v[i, pl.ds(16 * j, 16)] = jnp.zeros((16,), _F32)
        return carry
    with jax.named_scope("seg_zero"):
        lax.fori_loop(0, _BLK, zrow, None)
        row0 = sid * _RFULL

        @pl.when(sid < 15)
        def _():
            for k in range(_RFULL // _BLK):
                pltpu.sync_copy(rows_v,
                                agg_sh.at[pl.ds(row0 + k * _BLK, _BLK)])
            rem = _RFULL % _BLK
            pltpu.sync_copy(rows_v.at[pl.ds(0, rem)],
                            agg_sh.at[pl.ds(row0 + _RFULL - rem, rem)])

        @pl.when(sid == 15)
        def _():
            for k in range(_RLAST // _BLK):
                pltpu.sync_copy(rows_v,
                                agg_sh.at[pl.ds(row0 + k * _BLK, _BLK)])
            rem = _RLAST % _BLK
            pltpu.sync_copy(rows_v.at[pl.ds(0, rem)],
                            agg_sh.at[pl.ds(row0 + _RLAST - rem, rem)])

        pltpu.make_async_copy(src_hbm.at[wid], srcs_v, semi0).wait()

    bufs = ((rows0, dst0, ew0, semg0, semi0), (rows1, dst1, ew1, semg1, semi1))

    def _fetch(b, k):
        rows, dstb, ewb, semg, semi = bufs[k]
        pltpu.async_copy(m_hbm.at[srcs_v.at[b]], rows, semg)
        pltpu.async_copy(dst_hbm.at[wid, pl.ds(b, 1)], dstb, semi)
        pltpu.async_copy(ew_hbm.at[wid, pl.ds(b, 1)], ewb, semi)

    def _process(b, k):
        rows, dstb, ewb, semg, semi = bufs[k]
        pltpu.make_async_copy(m_hbm.at[srcs_v.at[b]], rows, semg).wait()
        pltpu.make_async_copy(dst_hbm.at[wid, pl.ds(b, 1)], dstb, semi).wait()
        pltpu.make_async_copy(ew_hbm.at[wid, pl.ds(b, 1)], ewb, semi).wait()

        def scale(g, c2):
            wv = ewb[0, pl.ds(g * 16, 16)]
            for i in range(16):
                w = jnp.broadcast_to(wv[i], (16,))
                r = g * 16 + i
                for j in range(8):
                    sl = pl.ds(16 * j, 16)
                    rows[r, sl] = rows[r, sl] * w
            return c2
        lax.fori_loop(0, _BLK // 16, scale, None)

        pltpu.sync_copy(rows, agg_sh.at[dstb.at[0]], add=True)

    with jax.named_scope("seg_prime"):
        _fetch(0, 0)
        _fetch(1, 1)
        plsc.subcore_barrier()

    def pair(t, carry):
        b = 2 * t
        _process(b, 0)

        @pl.when(b + 2 < nblk)
        def _():
            _fetch(b + 2, 0)

        _process(b + 1, 1)

        @pl.when(b + 3 < nblk)
        def _():
            _fetch(b + 3, 1)

        return carry

    with jax.named_scope("seg_main"):
        lax.fori_loop(0, (nblk - 1) // 2, pair, None)
        _process(nblk - 1, 0)
    with jax.named_scope("seg_tail"):
        plsc.subcore_barrier()

    @pl.when(sid < 15)
    def _():
        pltpu.sync_copy(agg_sh.at[pl.ds(row0, _RFULL)],
                        out_hbm.at[cid, pl.ds(row0, _RFULL)])

    @pl.when(sid == 15)
    def _():
        pltpu.sync_copy(agg_sh.at[pl.ds(row0, _RLAST)],
                        out_hbm.at[cid, pl.ds(row0, _RLAST)])


_seg = functools.partial(
    pl.kernel,
    out_type=jax.ShapeDtypeStruct((2, _N, _H), _F32),
    scratch_types=[
        pltpu.VMEM((_NBMAX, _BLK), jnp.int32),
        pltpu.VMEM((1, _BLK), jnp.int32),
        pltpu.VMEM((1, _BLK), jnp.int32),
        pltpu.VMEM((1, _BLK), _F32),
        pltpu.VMEM((1, _BLK), _F32),
        pltpu.VMEM((_BLK, _H), _F32),
        pltpu.VMEM((_BLK, _H), _F32),
        pltpu.VMEM_SHARED((_N, _H), _F32),
        pltpu.SemaphoreType.DMA,
        pltpu.SemaphoreType.DMA,
        pltpu.SemaphoreType.DMA,
        pltpu.SemaphoreType.DMA,
    ],
    mesh=plsc.VectorSubcoreMesh(core_axis_name="c", subcore_axis_name="s"),
)(_seg_body)


_RB = 2000
_G = _N // _RB


def _dotT(a, w):
    return lax.dot_general(a, w, (((1,), (1,)), ((), ())),
                           precision=_HI, preferred_element_type=_F32)


def _blk(shape, idx):
    return pl.BlockSpec(shape, idx)


def _pre_body(x_ref, w_ref, whh_ref, bhh_ref, m_ref, gh_ref):
    xv = x_ref[...]
    m_ref[...] = lax.dot_general(xv, w_ref[...], (((1,), (0,)), ((), ())),
                                 precision=_HI, preferred_element_type=_F32)
    gh_ref[...] = _dotT(xv, whh_ref[...]) + bhh_ref[...]


_pre = pl.pallas_call(
    _pre_body,
    grid=(_G,),
    in_specs=[_blk((_RB, _H), lambda i: (i, 0)),
              _blk((_H, _H), lambda i: (0, 0)),
              _blk((3 * _H, _H), lambda i: (0, 0)),
              _blk((1, 3 * _H), lambda i: (0, 0))],
    out_specs=[_blk((_RB, _H), lambda i: (i, 0)),
               _blk((_RB, 3 * _H), lambda i: (i, 0))],
    out_shape=[jax.ShapeDtypeStruct((_N, _H), _F32),
               jax.ShapeDtypeStruct((_N, 3 * _H), _F32)],
)


def _gru_body(parts_ref, gh_ref, xin_ref, wih_ref, bih_ref, hr_ref,
              stats_ref):
    agg = parts_ref[0] + parts_ref[1]
    gi = _dotT(agg, wih_ref[...]) + bih_ref[...]
    gh = gh_ref[...]
    r = jax.nn.sigmoid(gi[:, :_H] + gh[:, :_H])
    z = jax.nn.sigmoid(gi[:, _H:2 * _H] + gh[:, _H:2 * _H])
    n = jnp.tanh(gi[:, 2 * _H:] + r * gh[:, 2 * _H:])
    h = (1.0 - z) * n + z * xin_ref[...]
    h = jnp.maximum(h, 0.0)
    hr_ref[...] = h

    @pl.when(pl.program_id(0) == 0)
    def _():
        stats_ref[...] = jnp.zeros((2, _H), _F32)

    s1 = jnp.sum(h, axis=0, keepdims=True)
    s2 = jnp.sum(h * h, axis=0, keepdims=True)
    stats_ref[...] += jnp.concatenate([s1, s2], axis=0)


_gru = pl.pallas_call(
    _gru_body,
    grid=(_G,),
    in_specs=[_blk((2, _RB, _H), lambda i: (0, i, 0)),
              _blk((_RB, 3 * _H), lambda i: (i, 0)),
              _blk((_RB, _H), lambda i: (i, 0)),
              _blk((3 * _H, _H), lambda i: (0, 0)),
              _blk((1, 3 * _H), lambda i: (0, 0))],
    out_specs=[_blk((_RB, _H), lambda i: (i, 0)),
               _blk((2, _H), lambda i: (0, 0))],
    out_shape=[jax.ShapeDtypeStruct((_N, _H), _F32),
               jax.ShapeDtypeStruct((2, _H), _F32)],
)


def _bn_from_stats(h, stats_ref, g_ref, b_ref):
    mu = stats_ref[0:1, :] * (1.0 / _N)
    var = stats_ref[1:2, :] * (1.0 / _N) - mu * mu
    return (h - mu) / jnp.sqrt(var + 1e-5) * g_ref[...] + b_ref[...]


def _bnpre_body(hr_ref, stats_ref, g_ref, b_ref, w_ref, whh_ref, bhh_ref,
                h1_ref, m_ref, gh_ref):
    h1 = _bn_from_stats(hr_ref[...], stats_ref, g_ref, b_ref)
    h1_ref[...] = h1
    m_ref[...] = lax.dot_general(h1, w_ref[...], (((1,), (0,)), ((), ())),
                                 precision=_HI, preferred_element_type=_F32)
    gh_ref[...] = _dotT(h1, whh_ref[...]) + bhh_ref[...]


_bnpre = pl.pallas_call(
    _bnpre_body,
    grid=(_G,),
    in_specs=[_blk((_RB, _H), lambda i: (i, 0)),
              _blk((2, _H), lambda i: (0, 0)),
              _blk((1, _H), lambda i: (0, 0)),
              _blk((1, _H), lambda i: (0, 0)),
              _blk((_H, _H), lambda i: (0, 0)),
              _blk((3 * _H, _H), lambda i: (0, 0)),
              _blk((1, 3 * _H), lambda i: (0, 0))],
    out_specs=[_blk((_RB, _H), lambda i: (i, 0)),
               _blk((_RB, _H), lambda i: (i, 0)),
               _blk((_RB, 3 * _H), lambda i: (i, 0))],
    out_shape=[jax.ShapeDtypeStruct((_N, _H), _F32),
               jax.ShapeDtypeStruct((_N, _H), _F32),
               jax.ShapeDtypeStruct((_N, 3 * _H), _F32)],
)


def _bnlin_body(hr_ref, stats_ref, g_ref, b_ref, l1w_ref, l1b_ref, t_ref,
                stats3_ref):
    h = _bn_from_stats(hr_ref[...], stats_ref, g_ref, b_ref)
    t = jnp.maximum(_dotT(h, l1w_ref[...]) + l1b_ref[...], 0.0)
    t_ref[...] = t

    @pl.when(pl.program_id(0) == 0)
    def _():
        stats3_ref[...] = jnp.zeros((2, _H), _F32)

    s1 = jnp.sum(t, axis=0, keepdims=True)
    s2 = jnp.sum(t * t, axis=0, keepdims=True)
    stats3_ref[...] += jnp.concatenate([s1, s2], axis=0)


_bnlin = pl.pallas_call(
    _bnlin_body,
    grid=(_G,),
    in_specs=[_blk((_RB, _H), lambda i: (i, 0)),
              _blk((2, _H), lambda i: (0, 0)),
              _blk((1, _H), lambda i: (0, 0)),
              _blk((1, _H), lambda i: (0, 0)),
              _blk((_H, _H), lambda i: (0, 0)),
              _blk((1, _H), lambda i: (0, 0))],
    out_specs=[_blk((_RB, _H), lambda i: (i, 0)),
               _blk((2, _H), lambda i: (0, 0))],
    out_shape=[jax.ShapeDtypeStruct((_N, _H), _F32),
               jax.ShapeDtypeStruct((2, _H), _F32)],
)


def _out_body(t_ref, stats_ref, g_ref, b_ref, l2w_ref, out_ref):
    h = _bn_from_stats(t_ref[...], stats_ref, g_ref, b_ref)
    out_ref[...] = _dotT(h, l2w_ref[...])


_outk = pl.pallas_call(
    _out_body,
    grid=(_G,),
    in_specs=[_blk((_RB, _H), lambda i: (i, 0)),
              _blk((2, _H), lambda i: (0, 0)),
              _blk((1, _H), lambda i: (0, 0)),
              _blk((1, _H), lambda i: (0, 0)),
              _blk((1, _H), lambda i: (0, 0))],
    out_specs=[_blk((_RB, 1), lambda i: (i, 0))],
    out_shape=[jax.ShapeDtypeStruct((_N, 1), _F32)],
)


def kernel(x, edge_index, edge_attr, W1, Wih1, Whh1, bih1, bhh1, W2, Wih2,
           Whh2, bih2, bhh2, lin1_W, lin1_b, lin2_W, lin2_b, bn1_g, bn1_b,
           bn2_g, bn2_b, bn3_g, bn3_b):
    def _slab(arr):
        n0 = 16 * _NB0 * _BLK
        n1 = 16 * _NB1 * _BLK
        e0 = arr[:n0].reshape(16, _NB0, _BLK)
        e1 = jnp.pad(arr[n0:], (0, n1 - (_E - n0))).reshape(16, _NB1, _BLK)
        e1 = jnp.pad(e1, ((0, 0), (0, _NBMAX - _NB1), (0, 0)))
        return jnp.concatenate([e0, e1], axis=0)

    src = _slab(edge_index[0])
    dst = _slab(edge_index[1])
    ew = _slab(edge_attr)

    m1, gh1 = _pre(x, W1, Whh1, bhh1.reshape(1, -1))
    parts1 = _seg(m1, src, dst, ew)
    hr1, stats1 = _gru(parts1, gh1, x, Wih1, bih1.reshape(1, -1))

    h1, m2, gh2 = _bnpre(hr1, stats1, bn1_g.reshape(1, -1),
                         bn1_b.reshape(1, -1), W2, Whh2,
                         bhh2.reshape(1, -1))
    parts2 = _seg(m2, src, dst, ew)
    hr2, stats2 = _gru(parts2, gh2, h1, Wih2, bih2.reshape(1, -1))

    t, stats3 = _bnlin(hr2, stats2, bn2_g.reshape(1, -1),
                       bn2_b.reshape(1, -1), lin1_W, lin1_b.reshape(1, -1))
    (out,) = _outk(t, stats3, bn3_g.reshape(1, -1), bn3_b.reshape(1, -1),
                   lin2_W)
    return out.reshape(-1) + lin2_b[0]


# speedup vs baseline: 8.6517x; 1.5558x over previous
import functools

import jax
import jax.numpy as jnp
from jax import lax
from jax.experimental import pallas as pl
from jax.experimental.pallas import tpu as pltpu
from jax.experimental.pallas import tpu_sc as plsc

_N = 10000
_E = 320000
_H = 128

_BLK = 128
_WORKERS = 32
_NB0 = 79
_NBMAX = _NB0
_EPW = _NB0 * _BLK
_RFULL = 632
_RLAST = _N - 15 * _RFULL

_F32 = jnp.float32
_HI = lax.Precision.HIGHEST


def _seg_body(m_hbm, src_hbm, dst_hbm, ew_hbm, out_hbm,
              srcs_v, dst0, dst1, ew0, ew1, rows0, rows1, agg_sh,
              semg0, semg1, semi0, semi1):
    cid = lax.axis_index("c")
    sid = lax.axis_index("s")
    wid = cid * 16 + sid
    nblk = _NB0

    with jax.named_scope("seg_slab"):
        pltpu.async_copy(src_hbm.at[wid], srcs_v, semi0)

    rows_v = rows0

    def zrow(i, carry):
        for j in range(8):
            rows_v[i, pl.ds(16 * j, 16)] = jnp.zeros((16,), _F32)
        return carry
    with jax.named_scope("seg_zero"):
        lax.fori_loop(0, _BLK, zrow, None)
        row0 = sid * _RFULL

        @pl.when(sid < 15)
        def _():
            for k in range(_RFULL // _BLK):
                pltpu.sync_copy(rows_v,
                                agg_sh.at[pl.ds(row0 + k * _BLK, _BLK)])
            rem = _RFULL % _BLK
            pltpu.sync_copy(rows_v.at[pl.ds(0, rem)],
                            agg_sh.at[pl.ds(row0 + _RFULL - rem, rem)])

        @pl.when(sid == 15)
        def _():
            for k in range(_RLAST // _BLK):
                pltpu.sync_copy(rows_v,
                                agg_sh.at[pl.ds(row0 + k * _BLK, _BLK)])
            rem = _RLAST % _BLK
            pltpu.sync_copy(rows_v.at[pl.ds(0, rem)],
                            agg_sh.at[pl.ds(row0 + _RLAST - rem, rem)])

        pltpu.make_async_copy(src_hbm.at[wid], srcs_v, semi0).wait()

    bufs = ((rows0, dst0, ew0, semg0, semi0), (rows1, dst1, ew1, semg1, semi1))

    def _fetch(b, k):
        rows, dstb, ewb, semg, semi = bufs[k]
        pltpu.async_copy(m_hbm.at[srcs_v.at[b]], rows, semg)
        pltpu.async_copy(dst_hbm.at[wid, pl.ds(b, 1)], dstb, semi)
        pltpu.async_copy(ew_hbm.at[wid, pl.ds(b, 1)], ewb, semi)

    def _process(b, k):
        rows, dstb, ewb, semg, semi = bufs[k]
        pltpu.make_async_copy(m_hbm.at[srcs_v.at[b]], rows, semg).wait()
        pltpu.make_async_copy(dst_hbm.at[wid, pl.ds(b, 1)], dstb, semi).wait()
        pltpu.make_async_copy(ew_hbm.at[wid, pl.ds(b, 1)], ewb, semi).wait()

        def scale(g, c2):
            wv = ewb[0, pl.ds(g * 16, 16)]
            for i in range(16):
                w = jnp.broadcast_to(wv[i], (16,))
                r = g * 16 + i
                for j in range(8):
                    sl = pl.ds(16 * j, 16)
                    rows[r, sl] = rows[r, sl] * w
            return c2
        lax.fori_loop(0, _BLK // 16, scale, None)

        pltpu.sync_copy(rows, agg_sh.at[dstb.at[0]], add=True)

    with jax.named_scope("seg_prime"):
        _fetch(0, 0)
        _fetch(1, 1)
        plsc.subcore_barrier()

    def pair(t, carry):
        b = 2 * t
        _process(b, 0)

        @pl.when(b + 2 < nblk)
        def _():
            _fetch(b + 2, 0)

        _process(b + 1, 1)

        @pl.when(b + 3 < nblk)
        def _():
            _fetch(b + 3, 1)

        return carry

    with jax.named_scope("seg_main"):
        lax.fori_loop(0, (nblk - 1) // 2, pair, None)
        _process(nblk - 1, 0)
    with jax.named_scope("seg_tail"):
        plsc.subcore_barrier()

    @pl.when(sid < 15)
    def _():
        pltpu.sync_copy(agg_sh.at[pl.ds(row0, _RFULL)],
                        out_hbm.at[cid, pl.ds(row0, _RFULL)])

    @pl.when(sid == 15)
    def _():
        pltpu.sync_copy(agg_sh.at[pl.ds(row0, _RLAST)],
                        out_hbm.at[cid, pl.ds(row0, _RLAST)])


_seg = functools.partial(
    pl.kernel,
    out_type=jax.ShapeDtypeStruct((2, _N, _H), _F32),
    scratch_types=[
        pltpu.VMEM((_NBMAX, _BLK), jnp.int32),
        pltpu.VMEM((1, _BLK), jnp.int32),
        pltpu.VMEM((1, _BLK), jnp.int32),
        pltpu.VMEM((1, _BLK), _F32),
        pltpu.VMEM((1, _BLK), _F32),
        pltpu.VMEM((_BLK, _H), _F32),
        pltpu.VMEM((_BLK, _H), _F32),
        pltpu.VMEM_SHARED((_N, _H), _F32),
        pltpu.SemaphoreType.DMA,
        pltpu.SemaphoreType.DMA,
        pltpu.SemaphoreType.DMA,
        pltpu.SemaphoreType.DMA,
    ],
    mesh=plsc.VectorSubcoreMesh(core_axis_name="c", subcore_axis_name="s"),
)(_seg_body)


_RB = 2000
_G = _N // _RB


def _dotT(a, w):
    return lax.dot_general(a, w, (((1,), (1,)), ((), ())),
                           precision=_HI, preferred_element_type=_F32)


def _blk(shape, idx):
    return pl.BlockSpec(shape, idx)


def _pre_body(x_ref, w_ref, whh_ref, bhh_ref, m_ref, gh_ref):
    xv = x_ref[...]
    m_ref[...] = lax.dot_general(xv, w_ref[...], (((1,), (0,)), ((), ())),
                                 precision=_HI, preferred_element_type=_F32)
    gh_ref[...] = _dotT(xv, whh_ref[...]) + bhh_ref[...]


_pre = pl.pallas_call(
    _pre_body,
    grid=(_G,),
    in_specs=[_blk((_RB, _H), lambda i: (i, 0)),
              _blk((_H, _H), lambda i: (0, 0)),
              _blk((3 * _H, _H), lambda i: (0, 0)),
              _blk((1, 3 * _H), lambda i: (0, 0))],
    out_specs=[_blk((_RB, _H), lambda i: (i, 0)),
               _blk((_RB, 3 * _H), lambda i: (i, 0))],
    out_shape=[jax.ShapeDtypeStruct((_N, _H), _F32),
               jax.ShapeDtypeStruct((_N, 3 * _H), _F32)],
)


def _gru_body(parts_ref, gh_ref, xin_ref, wih_ref, bih_ref, hr_ref,
              stats_ref):
    agg = parts_ref[0] + parts_ref[1]
    gi = _dotT(agg, wih_ref[...]) + bih_ref[...]
    gh = gh_ref[...]
    r = jax.nn.sigmoid(gi[:, :_H] + gh[:, :_H])
    z = jax.nn.sigmoid(gi[:, _H:2 * _H] + gh[:, _H:2 * _H])
    n = jnp.tanh(gi[:, 2 * _H:] + r * gh[:, 2 * _H:])
    h = (1.0 - z) * n + z * xin_ref[...]
    h = jnp.maximum(h, 0.0)
    hr_ref[...] = h

    @pl.when(pl.program_id(0) == 0)
    def _():
        stats_ref[...] = jnp.zeros((2, _H), _F32)

    s1 = jnp.sum(h, axis=0, keepdims=True)
    s2 = jnp.sum(h * h, axis=0, keepdims=True)
    stats_ref[...] += jnp.concatenate([s1, s2], axis=0)


_gru = pl.pallas_call(
    _gru_body,
    grid=(_G,),
    in_specs=[_blk((2, _RB, _H), lambda i: (0, i, 0)),
              _blk((_RB, 3 * _H), lambda i: (i, 0)),
              _blk((_RB, _H), lambda i: (i, 0)),
              _blk((3 * _H, _H), lambda i: (0, 0)),
              _blk((1, 3 * _H), lambda i: (0, 0))],
    out_specs=[_blk((_RB, _H), lambda i: (i, 0)),
               _blk((2, _H), lambda i: (0, 0))],
    out_shape=[jax.ShapeDtypeStruct((_N, _H), _F32),
               jax.ShapeDtypeStruct((2, _H), _F32)],
)


def _bn_from_stats(h, stats_ref, g_ref, b_ref):
    mu = stats_ref[0:1, :] * (1.0 / _N)
    var = stats_ref[1:2, :] * (1.0 / _N) - mu * mu
    return (h - mu) / jnp.sqrt(var + 1e-5) * g_ref[...] + b_ref[...]


def _bnpre_body(hr_ref, stats_ref, g_ref, b_ref, w_ref, whh_ref, bhh_ref,
                h1_ref, m_ref, gh_ref):
    h1 = _bn_from_stats(hr_ref[...], stats_ref, g_ref, b_ref)
    h1_ref[...] = h1
    m_ref[...] = lax.dot_general(h1, w_ref[...], (((1,), (0,)), ((), ())),
                                 precision=_HI, preferred_element_type=_F32)
    gh_ref[...] = _dotT(h1, whh_ref[...]) + bhh_ref[...]


_bnpre = pl.pallas_call(
    _bnpre_body,
    grid=(_G,),
    in_specs=[_blk((_RB, _H), lambda i: (i, 0)),
              _blk((2, _H), lambda i: (0, 0)),
              _blk((1, _H), lambda i: (0, 0)),
              _blk((1, _H), lambda i: (0, 0)),
              _blk((_H, _H), lambda i: (0, 0)),
              _blk((3 * _H, _H), lambda i: (0, 0)),
              _blk((1, 3 * _H), lambda i: (0, 0))],
    out_specs=[_blk((_RB, _H), lambda i: (i, 0)),
               _blk((_RB, _H), lambda i: (i, 0)),
               _blk((_RB, 3 * _H), lambda i: (i, 0))],
    out_shape=[jax.ShapeDtypeStruct((_N, _H), _F32),
               jax.ShapeDtypeStruct((_N, _H), _F32),
               jax.ShapeDtypeStruct((_N, 3 * _H), _F32)],
)


def _bnlin_body(hr_ref, stats_ref, g_ref, b_ref, l1w_ref, l1b_ref, t_ref,
                stats3_ref):
    h = _bn_from_stats(hr_ref[...], stats_ref, g_ref, b_ref)
    t = jnp.maximum(_dotT(h, l1w_ref[...]) + l1b_ref[...], 0.0)
    t_ref[...] = t

    @pl.when(pl.program_id(0) == 0)
    def _():
        stats3_ref[...] = jnp.zeros((2, _H), _F32)

    s1 = jnp.sum(t, axis=0, keepdims=True)
    s2 = jnp.sum(t * t, axis=0, keepdims=True)
    stats3_ref[...] += jnp.concatenate([s1, s2], axis=0)


_bnlin = pl.pallas_call(
    _bnlin_body,
    grid=(_G,),
    in_specs=[_blk((_RB, _H), lambda i: (i, 0)),
              _blk((2, _H), lambda i: (0, 0)),
              _blk((1, _H), lambda i: (0, 0)),
              _blk((1, _H), lambda i: (0, 0)),
              _blk((_H, _H), lambda i: (0, 0)),
              _blk((1, _H), lambda i: (0, 0))],
    out_specs=[_blk((_RB, _H), lambda i: (i, 0)),
               _blk((2, _H), lambda i: (0, 0))],
    out_shape=[jax.ShapeDtypeStruct((_N, _H), _F32),
               jax.ShapeDtypeStruct((2, _H), _F32)],
)


def _out_body(t_ref, stats_ref, g_ref, b_ref, l2w_ref, out_ref):
    h = _bn_from_stats(t_ref[...], stats_ref, g_ref, b_ref)
    out_ref[...] = _dotT(h, l2w_ref[...])


_outk = pl.pallas_call(
    _out_body,
    grid=(_G,),
    in_specs=[_blk((_RB, _H), lambda i: (i, 0)),
              _blk((2, _H), lambda i: (0, 0)),
              _blk((1, _H), lambda i: (0, 0)),
              _blk((1, _H), lambda i: (0, 0)),
              _blk((1, _H), lambda i: (0, 0))],
    out_specs=[_blk((_RB, 1), lambda i: (i, 0))],
    out_shape=[jax.ShapeDtypeStruct((_N, 1), _F32)],
)


def kernel(x, edge_index, edge_attr, W1, Wih1, Whh1, bih1, bhh1, W2, Wih2,
           Whh2, bih2, bhh2, lin1_W, lin1_b, lin2_W, lin2_b, bn1_g, bn1_b,
           bn2_g, bn2_b, bn3_g, bn3_b):
    def _slab(arr, spread_pad):
        per = _E // _WORKERS
        padw = _EPW - per
        real = arr.reshape(_WORKERS, per)
        if spread_pad:
            padv = (jnp.arange(_WORKERS * padw, dtype=arr.dtype)
                    % _N).reshape(_WORKERS, padw)
        else:
            padv = jnp.zeros((_WORKERS, padw), arr.dtype)
        return jnp.concatenate([real, padv],
                               axis=1).reshape(_WORKERS, _NBMAX, _BLK)

    src = _slab(edge_index[0], True)
    dst = _slab(edge_index[1], True)
    ew = _slab(edge_attr, False)

    m1, gh1 = _pre(x, W1, Whh1, bhh1.reshape(1, -1))
    parts1 = _seg(m1, src, dst, ew)
    hr1, stats1 = _gru(parts1, gh1, x, Wih1, bih1.reshape(1, -1))

    h1, m2, gh2 = _bnpre(hr1, stats1, bn1_g.reshape(1, -1),
                         bn1_b.reshape(1, -1), W2, Whh2,
                         bhh2.reshape(1, -1))
    parts2 = _seg(m2, src, dst, ew)
    hr2, stats2 = _gru(parts2, gh2, h1, Wih2, bih2.reshape(1, -1))

    t, stats3 = _bnlin(hr2, stats2, bn2_g.reshape(1, -1),
                       bn2_b.reshape(1, -1), lin1_W, lin1_b.reshape(1, -1))
    (out,) = _outk(t, stats3, bn3_g.reshape(1, -1), bn3_b.reshape(1, -1),
                   lin2_W)
    return out.reshape(-1) + lin2_b[0]


# speedup vs baseline: 9.4082x; 1.0874x over previous
import functools

import jax
import jax.numpy as jnp
from jax import lax
from jax.experimental import pallas as pl
from jax.experimental.pallas import tpu as pltpu
from jax.experimental.pallas import tpu_sc as plsc

_N = 10000
_E = 320000
_H = 128

_BLK = 128
_WORKERS = 32
_NB0 = 79
_NBMAX = _NB0
_EPW = _NB0 * _BLK
_RFULL = 632
_RLAST = _N - 15 * _RFULL

_F32 = jnp.float32
_HI = lax.Precision.DEFAULT


def _seg_body(m_hbm, src_hbm, dst_hbm, ew_hbm, out_hbm,
              srcs_v, dst0, dst1, ew0, ew1, rows0, rows1, agg_sh,
              semg0, semg1, semi0, semi1):
    cid = lax.axis_index("c")
    sid = lax.axis_index("s")
    wid = cid * 16 + sid
    nblk = _NB0

    with jax.named_scope("seg_slab"):
        pltpu.async_copy(src_hbm.at[wid], srcs_v, semi0)

    rows_v = rows0

    def zrow(i, carry):
        for j in range(8):
            rows_v[i, pl.ds(16 * j, 16)] = jnp.zeros((16,), _F32)
        return carry
    with jax.named_scope("seg_zero"):
        lax.fori_loop(0, _BLK, zrow, None)
        row0 = sid * _RFULL

        @pl.when(sid < 15)
        def _():
            for k in range(_RFULL // _BLK):
                pltpu.sync_copy(rows_v,
                                agg_sh.at[pl.ds(row0 + k * _BLK, _BLK)])
            rem = _RFULL % _BLK
            pltpu.sync_copy(rows_v.at[pl.ds(0, rem)],
                            agg_sh.at[pl.ds(row0 + _RFULL - rem, rem)])

        @pl.when(sid == 15)
        def _():
            for k in range(_RLAST // _BLK):
                pltpu.sync_copy(rows_v,
                                agg_sh.at[pl.ds(row0 + k * _BLK, _BLK)])
            rem = _RLAST % _BLK
            pltpu.sync_copy(rows_v.at[pl.ds(0, rem)],
                            agg_sh.at[pl.ds(row0 + _RLAST - rem, rem)])

        pltpu.make_async_copy(src_hbm.at[wid], srcs_v, semi0).wait()

    bufs = ((rows0, dst0, ew0, semg0, semi0), (rows1, dst1, ew1, semg1, semi1))

    def _fetch(b, k):
        rows, dstb, ewb, semg, semi = bufs[k]
        pltpu.async_copy(m_hbm.at[srcs_v.at[b]], rows, semg)
        pltpu.async_copy(dst_hbm.at[wid, pl.ds(b, 1)], dstb, semi)
        pltpu.async_copy(ew_hbm.at[wid, pl.ds(b, 1)], ewb, semi)

    def _process(b, k):
        rows, dstb, ewb, semg, semi = bufs[k]
        pltpu.make_async_copy(m_hbm.at[srcs_v.at[b]], rows, semg).wait()
        pltpu.make_async_copy(dst_hbm.at[wid, pl.ds(b, 1)], dstb, semi).wait()
        pltpu.make_async_copy(ew_hbm.at[wid, pl.ds(b, 1)], ewb, semi).wait()

        def scale(g, c2):
            wv = ewb[0, pl.ds(g * 16, 16)]
            for i in range(16):
                w = jnp.broadcast_to(wv[i], (16,))
                r = g * 16 + i
                for j in range(8):
                    sl = pl.ds(16 * j, 16)
                    rows[r, sl] = rows[r, sl] * w
            return c2
        lax.fori_loop(0, _BLK // 16, scale, None)

        pltpu.sync_copy(rows, agg_sh.at[dstb.at[0]], add=True)

    with jax.named_scope("seg_prime"):
        _fetch(0, 0)
        _fetch(1, 1)
        plsc.subcore_barrier()

    def pair(t, carry):
        b = 2 * t
        _process(b, 0)

        @pl.when(b + 2 < nblk)
        def _():
            _fetch(b + 2, 0)

        _process(b + 1, 1)

        @pl.when(b + 3 < nblk)
        def _():
            _fetch(b + 3, 1)

        return carry

    with jax.named_scope("seg_main"):
        lax.fori_loop(0, (nblk - 1) // 2, pair, None)
        _process(nblk - 1, 0)
    with jax.named_scope("seg_tail"):
        plsc.subcore_barrier()

    @pl.when(sid < 15)
    def _():
        pltpu.sync_copy(agg_sh.at[pl.ds(row0, _RFULL)],
                        out_hbm.at[cid, pl.ds(row0, _RFULL)])

    @pl.when(sid == 15)
    def _():
        pltpu.sync_copy(agg_sh.at[pl.ds(row0, _RLAST)],
                        out_hbm.at[cid, pl.ds(row0, _RLAST)])


_seg = functools.partial(
    pl.kernel,
    out_type=jax.ShapeDtypeStruct((2, _N, _H), _F32),
    scratch_types=[
        pltpu.VMEM((_NBMAX, _BLK), jnp.int32),
        pltpu.VMEM((1, _BLK), jnp.int32),
        pltpu.VMEM((1, _BLK), jnp.int32),
        pltpu.VMEM((1, _BLK), _F32),
        pltpu.VMEM((1, _BLK), _F32),
        pltpu.VMEM((_BLK, _H), _F32),
        pltpu.VMEM((_BLK, _H), _F32),
        pltpu.VMEM_SHARED((_N, _H), _F32),
        pltpu.SemaphoreType.DMA,
        pltpu.SemaphoreType.DMA,
        pltpu.SemaphoreType.DMA,
        pltpu.SemaphoreType.DMA,
    ],
    mesh=plsc.VectorSubcoreMesh(core_axis_name="c", subcore_axis_name="s"),
)(_seg_body)


_RB = 2000
_G = _N // _RB


def _dotT(a, w):
    return lax.dot_general(a, w, (((1,), (1,)), ((), ())),
                           precision=_HI, preferred_element_type=_F32)


def _blk(shape, idx):
    return pl.BlockSpec(shape, idx)


def _pre_body(x_ref, w_ref, whh_ref, bhh_ref, m_ref, gh_ref):
    xv = x_ref[...]
    m_ref[...] = lax.dot_general(xv, w_ref[...], (((1,), (0,)), ((), ())),
                                 precision=_HI, preferred_element_type=_F32)
    gh_ref[...] = _dotT(xv, whh_ref[...]) + bhh_ref[...]


_pre = pl.pallas_call(
    _pre_body,
    grid=(_G,),
    in_specs=[_blk((_RB, _H), lambda i: (i, 0)),
              _blk((_H, _H), lambda i: (0, 0)),
              _blk((3 * _H, _H), lambda i: (0, 0)),
              _blk((1, 3 * _H), lambda i: (0, 0))],
    out_specs=[_blk((_RB, _H), lambda i: (i, 0)),
               _blk((_RB, 3 * _H), lambda i: (i, 0))],
    out_shape=[jax.ShapeDtypeStruct((_N, _H), _F32),
               jax.ShapeDtypeStruct((_N, 3 * _H), _F32)],
)


def _gru_body(parts_ref, gh_ref, xin_ref, wih_ref, bih_ref, hr_ref,
              stats_ref):
    agg = parts_ref[0] + parts_ref[1]
    gi = _dotT(agg, wih_ref[...]) + bih_ref[...]
    gh = gh_ref[...]
    r = jax.nn.sigmoid(gi[:, :_H] + gh[:, :_H])
    z = jax.nn.sigmoid(gi[:, _H:2 * _H] + gh[:, _H:2 * _H])
    n = jnp.tanh(gi[:, 2 * _H:] + r * gh[:, 2 * _H:])
    h = (1.0 - z) * n + z * xin_ref[...]
    h = jnp.maximum(h, 0.0)
    hr_ref[...] = h

    @pl.when(pl.program_id(0) == 0)
    def _():
        stats_ref[...] = jnp.zeros((2, _H), _F32)

    s1 = jnp.sum(h, axis=0, keepdims=True)
    s2 = jnp.sum(h * h, axis=0, keepdims=True)
    stats_ref[...] += jnp.concatenate([s1, s2], axis=0)


_gru = pl.pallas_call(
    _gru_body,
    grid=(_G,),
    in_specs=[_blk((2, _RB, _H), lambda i: (0, i, 0)),
              _blk((_RB, 3 * _H), lambda i: (i, 0)),
              _blk((_RB, _H), lambda i: (i, 0)),
              _blk((3 * _H, _H), lambda i: (0, 0)),
              _blk((1, 3 * _H), lambda i: (0, 0))],
    out_specs=[_blk((_RB, _H), lambda i: (i, 0)),
               _blk((2, _H), lambda i: (0, 0))],
    out_shape=[jax.ShapeDtypeStruct((_N, _H), _F32),
               jax.ShapeDtypeStruct((2, _H), _F32)],
)


def _bn_from_stats(h, stats_ref, g_ref, b_ref):
    mu = stats_ref[0:1, :] * (1.0 / _N)
    var = stats_ref[1:2, :] * (1.0 / _N) - mu * mu
    return (h - mu) / jnp.sqrt(var + 1e-5) * g_ref[...] + b_ref[...]


def _bnpre_body(hr_ref, stats_ref, g_ref, b_ref, w_ref, whh_ref, bhh_ref,
                h1_ref, m_ref, gh_ref):
    h1 = _bn_from_stats(hr_ref[...], stats_ref, g_ref, b_ref)
    h1_ref[...] = h1
    m_ref[...] = lax.dot_general(h1, w_ref[...], (((1,), (0,)), ((), ())),
                                 precision=_HI, preferred_element_type=_F32)
    gh_ref[...] = _dotT(h1, whh_ref[...]) + bhh_ref[...]


_bnpre = pl.pallas_call(
    _bnpre_body,
    grid=(_G,),
    in_specs=[_blk((_RB, _H), lambda i: (i, 0)),
              _blk((2, _H), lambda i: (0, 0)),
              _blk((1, _H), lambda i: (0, 0)),
              _blk((1, _H), lambda i: (0, 0)),
              _blk((_H, _H), lambda i: (0, 0)),
              _blk((3 * _H, _H), lambda i: (0, 0)),
              _blk((1, 3 * _H), lambda i: (0, 0))],
    out_specs=[_blk((_RB, _H), lambda i: (i, 0)),
               _blk((_RB, _H), lambda i: (i, 0)),
               _blk((_RB, 3 * _H), lambda i: (i, 0))],
    out_shape=[jax.ShapeDtypeStruct((_N, _H), _F32),
               jax.ShapeDtypeStruct((_N, _H), _F32),
               jax.ShapeDtypeStruct((_N, 3 * _H), _F32)],
)


def _bnlin_body(hr_ref, stats_ref, g_ref, b_ref, l1w_ref, l1b_ref, t_ref,
                stats3_ref):
    h = _bn_from_stats(hr_ref[...], stats_ref, g_ref, b_ref)
    t = jnp.maximum(_dotT(h, l1w_ref[...]) + l1b_ref[...], 0.0)
    t_ref[...] = t

    @pl.when(pl.program_id(0) == 0)
    def _():
        stats3_ref[...] = jnp.zeros((2, _H), _F32)

    s1 = jnp.sum(t, axis=0, keepdims=True)
    s2 = jnp.sum(t * t, axis=0, keepdims=True)
    stats3_ref[...] += jnp.concatenate([s1, s2], axis=0)


_bnlin = pl.pallas_call(
    _bnlin_body,
    grid=(_G,),
    in_specs=[_blk((_RB, _H), lambda i: (i, 0)),
              _blk((2, _H), lambda i: (0, 0)),
              _blk((1, _H), lambda i: (0, 0)),
              _blk((1, _H), lambda i: (0, 0)),
              _blk((_H, _H), lambda i: (0, 0)),
              _blk((1, _H), lambda i: (0, 0))],
    out_specs=[_blk((_RB, _H), lambda i: (i, 0)),
               _blk((2, _H), lambda i: (0, 0))],
    out_shape=[jax.ShapeDtypeStruct((_N, _H), _F32),
               jax.ShapeDtypeStruct((2, _H), _F32)],
)


def _out_body(t_ref, stats_ref, g_ref, b_ref, l2w_ref, out_ref):
    h = _bn_from_stats(t_ref[...], stats_ref, g_ref, b_ref)
    out_ref[...] = _dotT(h, l2w_ref[...])


_outk = pl.pallas_call(
    _out_body,
    grid=(_G,),
    in_specs=[_blk((_RB, _H), lambda i: (i, 0)),
              _blk((2, _H), lambda i: (0, 0)),
              _blk((1, _H), lambda i: (0, 0)),
              _blk((1, _H), lambda i: (0, 0)),
              _blk((1, _H), lambda i: (0, 0))],
    out_specs=[_blk((_RB, 1), lambda i: (i, 0))],
    out_shape=[jax.ShapeDtypeStruct((_N, 1), _F32)],
)


def kernel(x, edge_index, edge_attr, W1, Wih1, Whh1, bih1, bhh1, W2, Wih2,
           Whh2, bih2, bhh2, lin1_W, lin1_b, lin2_W, lin2_b, bn1_g, bn1_b,
           bn2_g, bn2_b, bn3_g, bn3_b):
    def _slab(arr, spread_pad):
        per = _E // _WORKERS
        padw = _EPW - per
        real = arr.reshape(_WORKERS, per)
        if spread_pad:
            padv = (jnp.arange(_WORKERS * padw, dtype=arr.dtype)
                    % _N).reshape(_WORKERS, padw)
        else:
            padv = jnp.zeros((_WORKERS, padw), arr.dtype)
        return jnp.concatenate([real, padv],
                               axis=1).reshape(_WORKERS, _NBMAX, _BLK)

    src = _slab(edge_index[0], True)
    dst = _slab(edge_index[1], True)
    ew = _slab(edge_attr, False)

    m1, gh1 = _pre(x, W1, Whh1, bhh1.reshape(1, -1))
    parts1 = _seg(m1, src, dst, ew)
    hr1, stats1 = _gru(parts1, gh1, x, Wih1, bih1.reshape(1, -1))

    h1, m2, gh2 = _bnpre(hr1, stats1, bn1_g.reshape(1, -1),
                         bn1_b.reshape(1, -1), W2, Whh2,
                         bhh2.reshape(1, -1))
    parts2 = _seg(m2, src, dst, ew)
    hr2, stats2 = _gru(parts2, gh2, h1, Wih2, bih2.reshape(1, -1))

    t, stats3 = _bnlin(hr2, stats2, bn2_g.reshape(1, -1),
                       bn2_b.reshape(1, -1), lin1_W, lin1_b.reshape(1, -1))
    (out,) = _outk(t, stats3, bn3_g.reshape(1, -1), bn3_b.reshape(1, -1),
                   lin2_W)
    return out.reshape(-1) + lin2_b[0]
